# baseline (speedup 1.0000x reference)
import functools

import jax
import jax.numpy as jnp
from jax.experimental import pallas as pl
from jax.experimental.pallas import tpu as pltpu
from jax.experimental.pallas import tpu_sc as plsc


def kernel(output, encoder_outputs, encoder_sequence_lengths):
    del output
    B, T, D = encoder_outputs.shape
    flat = encoder_outputs.reshape(B * T, D)
    lengths = jnp.asarray(encoder_sequence_lengths, jnp.int32)

    mesh = plsc.ScalarSubcoreMesh(axis_name="c", num_cores=1)

    @functools.partial(
        pl.kernel,
        mesh=mesh,
        out_type=jax.ShapeDtypeStruct((B, D), jnp.float32),
        scratch_types=[
            pltpu.SMEM((B,), jnp.int32),
            pltpu.SemaphoreType.DMA,
        ],
    )
    def gather_last(table_hbm, len_hbm, out_hbm, len_s, sem):
        pltpu.sync_copy(len_hbm, len_s)
        for b in range(B):
            idx = len_s[b] - 1 + b * T
            pltpu.async_copy(
                table_hbm.at[pl.ds(idx, 1)], out_hbm.at[pl.ds(b, 1)], sem
            )
        pltpu.make_async_copy(table_hbm.at[pl.ds(0, B)], out_hbm, sem).wait()

    return gather_last(flat, lengths)
